# baseline (speedup 1.0000x reference)
import functools

import jax
import jax.numpy as jnp
from jax.experimental import pallas as pl
from jax.experimental.pallas import tpu as pltpu
from jax.experimental.pallas import tpu_sc as plsc

H = 2048
EMB = 128
CHUNK = 64
OVERLAP = 16
TOPK = 3
STRIDE = CHUNK - OVERLAP

_POOL_ROWS = 64

_INV_SQRT2 = 0.7071067811865476


def _gelu_exact(x):
    return 0.5 * x * (1.0 + jax.lax.erf(x * _INV_SQRT2))


def _retrieve_body(hs_ref, wq_ref, bq_ref, w1_ref, b1_ref, w2_ref, b2_ref,
                   lng_ref, lnb_ref, wf1b_ref, wgb_ref,
                   idx_ref, wexp_ref, vproj_ref, gproj_ref, x_sc, *,
                   n_chunks, batch, seq_len, seq_tile):
    b, t = pl.program_id(0), pl.program_id(1)
    n_t = pl.num_programs(1)
    hsb = hs_ref[0]
    c = jax.lax.broadcasted_iota(jnp.int32, (_POOL_ROWS, seq_tile), 0)
    s = jax.lax.broadcasted_iota(jnp.int32, (_POOL_ROWS, seq_tile), 1)
    s = s + t * seq_tile
    in_win = (s >= c * STRIDE) & (s < c * STRIDE + CHUNK) & (c < n_chunks)
    pmat = jnp.where(in_win, 1.0 / CHUNK, 0.0)
    pmat = pmat + jnp.where(c == n_chunks, 1.0 / seq_len, 0.0)
    part = jnp.dot(pmat, hsb, preferred_element_type=jnp.float32)

    @pl.when(t == 0)
    def _init():
        x_sc[b] = part

    @pl.when(t != 0)
    def _acc():
        x_sc[b] += part

    @pl.when((b == batch - 1) & (t == n_t - 1))
    def _retrieve():
        _retrieve_tail(wq_ref, bq_ref, w1_ref, b1_ref, w2_ref, b2_ref,
                       lng_ref, lnb_ref, wf1b_ref, wgb_ref,
                       idx_ref, wexp_ref, vproj_ref, gproj_ref, x_sc,
                       n_chunks=n_chunks, batch=batch)


def _retrieve_tail(wq_ref, bq_ref, w1_ref, b1_ref, w2_ref, b2_ref,
                   lng_ref, lnb_ref, wf1b_ref, wgb_ref,
                   idx_ref, wexp_ref, vproj_ref, gproj_ref, x_sc, *,
                   n_chunks, batch):
    X = x_sc[...]
    rows = batch * _POOL_ROWS
    Xf = X.reshape(rows, H)
    hmean = X[:, n_chunks, :]
    qpool = jnp.dot(hmean, wq_ref[...], preferred_element_type=jnp.float32)
    qpool = qpool + bq_ref[...]
    row_i = jax.lax.broadcasted_iota(jnp.int32, (rows, H), 0)
    qexp = jnp.broadcast_to(qpool[:, None, :], (batch, _POOL_ROWS, H))
    Xe = jnp.where((row_i % _POOL_ROWS) == n_chunks, qexp.reshape(rows, H), Xf)
    h = jnp.dot(Xe, w1_ref[...], preferred_element_type=jnp.float32)
    h = _gelu_exact(h + b1_ref[...])
    e = jnp.dot(h, w2_ref[...], preferred_element_type=jnp.float32) + b2_ref[...]
    mu = jnp.mean(e, axis=-1, keepdims=True)
    var = jnp.mean((e - mu) * (e - mu), axis=-1, keepdims=True)
    e = (e - mu) * jax.lax.rsqrt(var + 1e-5) * lng_ref[...] + lnb_ref[...]
    nrm = jnp.maximum(jnp.sqrt(jnp.sum(e * e, axis=-1, keepdims=True)), 1e-12)
    en = e / nrm
    qe = en.reshape(batch, _POOL_ROWS, EMB)[:, n_chunks, :]
    sim = jax.lax.dot_general(qe, en, (((1,), (1,)), ((), ())),
                              preferred_element_type=jnp.float32)
    col = jax.lax.broadcasted_iota(jnp.int32, (batch, rows), 1)
    colf = col.astype(jnp.float32)
    s = jnp.where((col % _POOL_ROWS) < n_chunks, sim, -1e30)
    scores, onehots = [], []
    for _ in range(TOPK):
        m = jnp.max(s, axis=1, keepdims=True)
        first = jnp.min(jnp.where(s >= m, colf, 1e9), axis=1, keepdims=True)
        oh = (colf == first).astype(jnp.float32)
        scores.append(m)
        onehots.append(oh)
        s = jnp.where(oh > 0.0, -1e30, s)
    es = [jnp.exp(sc - scores[0]) for sc in scores]
    z = es[0] + es[1] + es[2]
    k16 = jax.lax.broadcasted_iota(jnp.int32, (batch, 16), 1)
    idxm = jnp.zeros((batch, 16), jnp.float32)
    wmat = jnp.zeros((batch, 16), jnp.float32)
    for k in range(TOPK):
        first_k = jnp.min(jnp.where(onehots[k] > 0.0, colf, 1e9), axis=1,
                          keepdims=True)
        idxm = jnp.where(k16 == k, first_k, idxm)
        wmat = jnp.where(k16 == k, es[k] / z, wmat)
    idx_ref[...] = idxm.astype(jnp.int32)
    wexp_ref[...] = jnp.broadcast_to(wmat[:, :, None], (batch, 16, 16))
    vproj = jnp.dot(Xf.astype(jnp.bfloat16),
                    wf1b_ref[...].astype(jnp.bfloat16),
                    preferred_element_type=jnp.float32)
    vproj_ref[...] = vproj
    gp = jnp.sum(Xf * wgb_ref[...], axis=1, keepdims=True)
    gproj_ref[...] = jnp.broadcast_to(gp, (rows, 128))


def _sc_gather_body(vproj_hbm, gproj_hbm, idx_hbm, wexp_hbm, f1_hbm, gadd_hbm,
                    idx_vmem, wexp_vmem, rows_vmem, grows_vmem, acc_vmem,
                    gacc_vmem, *, batch):
    c = jax.lax.axis_index("c")
    s = jax.lax.axis_index("s")

    @pl.when((c == 0) & (s < batch))
    def _():
        pltpu.sync_copy(idx_hbm.at[s], idx_vmem)
        pltpu.sync_copy(wexp_hbm.at[s], wexp_vmem)
        pltpu.sync_copy(vproj_hbm.at[idx_vmem], rows_vmem)
        pltpu.sync_copy(gproj_hbm.at[idx_vmem], grows_vmem)

        @pl.loop(0, H, step=16)
        def _f1(c0):
            sl = pl.ds(c0, 16)
            acc = wexp_vmem[0, :] * rows_vmem[0, sl]
            for k in range(1, TOPK):
                acc += wexp_vmem[k, :] * rows_vmem[k, sl]
            acc_vmem[sl] = acc

        @pl.loop(0, 128, step=16)
        def _g(c0):
            sl = pl.ds(c0, 16)
            acc = wexp_vmem[0, :] * grows_vmem[0, sl]
            for k in range(1, TOPK):
                acc += wexp_vmem[k, :] * grows_vmem[k, sl]
            gacc_vmem[sl] = acc

        pltpu.sync_copy(acc_vmem, f1_hbm.at[s])
        pltpu.sync_copy(gacc_vmem, gadd_hbm.at[s])


def _fusion_body(hs_ref, f1_ref, bf1_ref, gadd_ref, wgt_ref, wf1_ref, wf2_ref,
                 bf2_ref, bg_ref, out_ref, wf1bf_ref):
    b, t = pl.program_id(0), pl.program_id(1)

    @pl.when((b == 0) & (t == 0))
    def _cast_weights():
        wf1bf_ref[...] = wf1_ref[...].astype(jnp.bfloat16)

    x = hs_ref[0]
    a = jnp.dot(x.astype(jnp.bfloat16), wf1bf_ref[...],
                preferred_element_type=jnp.float32)
    a = a + f1_ref[0] + bf1_ref[...]
    hgelu = _gelu_exact(a)
    f = jnp.dot(hgelu.astype(jnp.bfloat16), wf2_ref[...],
                preferred_element_type=jnp.float32) + bf2_ref[...]
    gl = (jnp.sum(x * wgt_ref[...], axis=1, keepdims=True)
          + gadd_ref[0][0, 0] + bg_ref[0, 0])
    out_ref[0] = x + jax.nn.sigmoid(gl) * f


def kernel(hidden_states, W1, b1, W2, b2, ln_g, ln_b, Wq, bq, Wg, bg,
           Wf1, bf1, Wf2, bf2):
    B, S, Hd = hidden_states.shape
    n_chunks = (S - CHUNK) // STRIDE + 1
    f32 = jnp.float32

    rows = B * _POOL_ROWS

    PT = 512
    idx16, wexp, vproj, gproj = pl.pallas_call(
        functools.partial(_retrieve_body, n_chunks=n_chunks, batch=B,
                          seq_len=S, seq_tile=PT),
        grid=(B, S // PT),
        in_specs=[
            pl.BlockSpec((1, PT, Hd), lambda b, t: (b, t, 0)),
            pl.BlockSpec((Hd, Hd), lambda b, t: (0, 0)),
            pl.BlockSpec((1, Hd), lambda b, t: (0, 0)),
            pl.BlockSpec((Hd, Hd // 2), lambda b, t: (0, 0)),
            pl.BlockSpec((1, Hd // 2), lambda b, t: (0, 0)),
            pl.BlockSpec((Hd // 2, EMB), lambda b, t: (0, 0)),
            pl.BlockSpec((1, EMB), lambda b, t: (0, 0)),
            pl.BlockSpec((1, EMB), lambda b, t: (0, 0)),
            pl.BlockSpec((1, EMB), lambda b, t: (0, 0)),
            pl.BlockSpec((Hd, Hd), lambda b, t: (1, 0)),
            pl.BlockSpec((1, Hd), lambda b, t: (0, 0)),
        ],
        out_specs=(pl.BlockSpec((B, 16), lambda b, t: (0, 0)),
                   pl.BlockSpec((B, 16, 16), lambda b, t: (0, 0, 0)),
                   pl.BlockSpec((rows, Hd), lambda b, t: (0, 0)),
                   pl.BlockSpec((rows, 128), lambda b, t: (0, 0))),
        out_shape=(jax.ShapeDtypeStruct((B, 16), jnp.int32),
                   jax.ShapeDtypeStruct((B, 16, 16), f32),
                   jax.ShapeDtypeStruct((rows, Hd), f32),
                   jax.ShapeDtypeStruct((rows, 128), f32)),
        scratch_shapes=[pltpu.VMEM((B, _POOL_ROWS, Hd), f32)],
    )(hidden_states, Wq, bq.reshape(1, Hd), W1, b1.reshape(1, -1), W2,
      b2.reshape(1, -1), ln_g.reshape(1, -1), ln_b.reshape(1, -1),
      Wf1, Wg[Hd:, 0].reshape(1, Hd))

    sc_kernel = pl.kernel(
        functools.partial(_sc_gather_body, batch=B),
        out_type=(jax.ShapeDtypeStruct((B, Hd), f32),
                  jax.ShapeDtypeStruct((B, 128), f32)),
        mesh=plsc.VectorSubcoreMesh(core_axis_name="c", subcore_axis_name="s"),
        scratch_types=[pltpu.VMEM((16,), jnp.int32),
                       pltpu.VMEM((16, 16), f32),
                       pltpu.VMEM((16, Hd), f32),
                       pltpu.VMEM((16, 128), f32),
                       pltpu.VMEM((Hd,), f32),
                       pltpu.VMEM((128,), f32)],
    )
    f1add, gadd = sc_kernel(vproj, gproj, idx16, wexp)

    TS = 512
    grid = (B, S // TS)
    out = pl.pallas_call(
        _fusion_body,
        grid=grid,
        in_specs=[
            pl.BlockSpec((1, TS, Hd), lambda b, t: (b, t, 0)),
            pl.BlockSpec((1, 1, Hd), lambda b, t: (b, 0, 0)),
            pl.BlockSpec((1, Hd), lambda b, t: (0, 0)),
            pl.BlockSpec((1, 1, 128), lambda b, t: (b, 0, 0)),
            pl.BlockSpec((1, Hd), lambda b, t: (0, 0)),
            pl.BlockSpec((Hd, Hd), lambda b, t: (0, 0)),
            pl.BlockSpec((Hd, Hd), lambda b, t: (0, 0)),
            pl.BlockSpec((1, Hd), lambda b, t: (0, 0)),
            pl.BlockSpec((1, 128), lambda b, t: (0, 0)),
        ],
        out_specs=pl.BlockSpec((1, TS, Hd), lambda b, t: (b, t, 0)),
        out_shape=jax.ShapeDtypeStruct((B, S, Hd), f32),
        scratch_shapes=[pltpu.VMEM((Hd, Hd), jnp.bfloat16)],
    )(hidden_states, f1add[:, None, :], bf1.reshape(1, Hd),
      gadd[:, None, :], Wg[:Hd, 0].reshape(1, Hd), Wf1,
      Wf2.astype(jnp.bfloat16), bf2.reshape(1, Hd),
      jnp.broadcast_to(bg.reshape(1, 1), (1, 128)))
    return out


# speedup vs baseline: 1.0198x; 1.0198x over previous
import functools

import jax
import jax.numpy as jnp
from jax.experimental import pallas as pl
from jax.experimental.pallas import tpu as pltpu
from jax.experimental.pallas import tpu_sc as plsc

H = 2048
EMB = 128
CHUNK = 64
OVERLAP = 16
TOPK = 3
STRIDE = CHUNK - OVERLAP

_POOL_ROWS = 64
_IDXW = 4
_PW = H + 128

_INV_SQRT2 = 0.7071067811865476


def _gelu_exact(x):
    return 0.5 * x * (1.0 + jax.lax.erf(x * _INV_SQRT2))


def _retrieve_body(hs_ref, wq_ref, bq_ref, w1_ref, b1_ref, w2_ref, b2_ref,
                   lng_ref, lnb_ref, wf1b_ref, wgb_ref,
                   idx_ref, wexp_ref, vproj_ref, x_sc, *,
                   n_chunks, batch, seq_len, seq_tile):
    b, t = pl.program_id(0), pl.program_id(1)
    n_t = pl.num_programs(1)
    hsb = hs_ref[0]
    c = jax.lax.broadcasted_iota(jnp.int32, (_POOL_ROWS, seq_tile), 0)
    s = jax.lax.broadcasted_iota(jnp.int32, (_POOL_ROWS, seq_tile), 1)
    s = s + t * seq_tile
    in_win = (s >= c * STRIDE) & (s < c * STRIDE + CHUNK) & (c < n_chunks)
    pmat = jnp.where(in_win, 1.0 / CHUNK, 0.0)
    pmat = pmat + jnp.where(c == n_chunks, 1.0 / seq_len, 0.0)
    part = jnp.dot(pmat, hsb, preferred_element_type=jnp.float32)

    @pl.when(t == 0)
    def _init():
        x_sc[b] = part

    @pl.when(t != 0)
    def _acc():
        x_sc[b] += part

    @pl.when((b == batch - 1) & (t == n_t - 1))
    def _retrieve():
        _retrieve_tail(wq_ref, bq_ref, w1_ref, b1_ref, w2_ref, b2_ref,
                       lng_ref, lnb_ref, wf1b_ref, wgb_ref,
                       idx_ref, wexp_ref, vproj_ref, x_sc,
                       n_chunks=n_chunks, batch=batch)


def _retrieve_tail(wq_ref, bq_ref, w1_ref, b1_ref, w2_ref, b2_ref,
                   lng_ref, lnb_ref, wf1b_ref, wgb_ref,
                   idx_ref, wexp_ref, vproj_ref, x_sc, *,
                   n_chunks, batch):
    X = x_sc[...]
    rows = batch * _POOL_ROWS
    Xf = X.reshape(rows, H)
    hmean = X[:, n_chunks, :]
    qpool = jnp.dot(hmean, wq_ref[...], preferred_element_type=jnp.float32)
    qpool = qpool + bq_ref[...]
    row_i = jax.lax.broadcasted_iota(jnp.int32, (rows, H), 0)
    qexp = jnp.broadcast_to(qpool[:, None, :], (batch, _POOL_ROWS, H))
    Xe = jnp.where((row_i % _POOL_ROWS) == n_chunks, qexp.reshape(rows, H), Xf)
    h = jnp.dot(Xe, w1_ref[...], preferred_element_type=jnp.float32)
    h = _gelu_exact(h + b1_ref[...])
    e = jnp.dot(h, w2_ref[...], preferred_element_type=jnp.float32) + b2_ref[...]
    mu = jnp.mean(e, axis=-1, keepdims=True)
    var = jnp.mean((e - mu) * (e - mu), axis=-1, keepdims=True)
    e = (e - mu) * jax.lax.rsqrt(var + 1e-5) * lng_ref[...] + lnb_ref[...]
    nrm = jnp.maximum(jnp.sqrt(jnp.sum(e * e, axis=-1, keepdims=True)), 1e-12)
    en = e / nrm
    qe = en.reshape(batch, _POOL_ROWS, EMB)[:, n_chunks, :]
    sim = jax.lax.dot_general(qe, en, (((1,), (1,)), ((), ())),
                              preferred_element_type=jnp.float32)
    col = jax.lax.broadcasted_iota(jnp.int32, (batch, rows), 1)
    colf = col.astype(jnp.float32)
    s = jnp.where((col % _POOL_ROWS) < n_chunks, sim, -1e30)
    scores, onehots = [], []
    for _ in range(TOPK):
        m = jnp.max(s, axis=1, keepdims=True)
        first = jnp.min(jnp.where(s >= m, colf, 1e9), axis=1, keepdims=True)
        oh = (colf == first).astype(jnp.float32)
        scores.append(m)
        onehots.append(oh)
        s = jnp.where(oh > 0.0, -1e30, s)
    es = [jnp.exp(sc - scores[0]) for sc in scores]
    z = es[0] + es[1] + es[2]
    k16 = jax.lax.broadcasted_iota(jnp.int32, (batch, _IDXW), 1)
    idxm = jnp.zeros((batch, _IDXW), jnp.float32)
    wmat = jnp.zeros((batch, _IDXW), jnp.float32)
    for k in range(TOPK):
        first_k = jnp.min(jnp.where(onehots[k] > 0.0, colf, 1e9), axis=1,
                          keepdims=True)
        idxm = jnp.where(k16 == k, first_k, idxm)
        wmat = jnp.where(k16 == k, es[k] / z, wmat)
    idx_ref[...] = idxm.astype(jnp.int32)
    wexp_ref[...] = jnp.broadcast_to(wmat[:, :, None], (batch, _IDXW, 16))
    vproj = jnp.dot(Xf.astype(jnp.bfloat16),
                    wf1b_ref[...].astype(jnp.bfloat16),
                    preferred_element_type=jnp.float32)
    vproj_ref[:, :H] = vproj
    gp = jnp.sum(Xf * wgb_ref[...], axis=1, keepdims=True)
    vproj_ref[:, H:] = jnp.broadcast_to(gp, (rows, 128))


def _sc_gather_body(vproj_hbm, idx_hbm, wexp_hbm, f1_hbm, gadd_hbm,
                    idx_vmem, wexp_vmem, rows_vmem, acc_vmem, *, batch):
    c = jax.lax.axis_index("c")
    s = jax.lax.axis_index("s")

    @pl.when((c == 0) & (s < batch))
    def _():
        pltpu.sync_copy(idx_hbm.at[s], idx_vmem)
        pltpu.sync_copy(wexp_hbm.at[s], wexp_vmem)
        pltpu.sync_copy(vproj_hbm.at[idx_vmem], rows_vmem)

        @pl.loop(0, H + 128, step=16)
        def _f1(c0):
            sl = pl.ds(c0, 16)
            acc = wexp_vmem[0, :] * rows_vmem[0, sl]
            for k in range(1, TOPK):
                acc += wexp_vmem[k, :] * rows_vmem[k, sl]
            acc_vmem[sl] = acc

        pltpu.sync_copy(acc_vmem.at[pl.ds(0, H)], f1_hbm.at[s])
        pltpu.sync_copy(acc_vmem.at[pl.ds(H, 128)], gadd_hbm.at[s])


def _fusion_body(hs_ref, f1_ref, bf1_ref, gadd_ref, wgt_ref, wf1_ref, wf2_ref,
                 bf2_ref, bg_ref, out_ref, wf1bf_ref):
    b, t = pl.program_id(0), pl.program_id(1)

    @pl.when((b == 0) & (t == 0))
    def _cast_weights():
        wf1bf_ref[...] = wf1_ref[...].astype(jnp.bfloat16)

    x = hs_ref[0]
    a = jnp.dot(x.astype(jnp.bfloat16), wf1bf_ref[...],
                preferred_element_type=jnp.float32)
    a = a + f1_ref[0] + bf1_ref[...]
    hgelu = _gelu_exact(a)
    f = jnp.dot(hgelu.astype(jnp.bfloat16), wf2_ref[...],
                preferred_element_type=jnp.float32) + bf2_ref[...]
    gl = (jnp.sum(x * wgt_ref[...], axis=1, keepdims=True)
          + gadd_ref[0][0, 0] + bg_ref[0, 0])
    out_ref[0] = x + jax.nn.sigmoid(gl) * f


def kernel(hidden_states, W1, b1, W2, b2, ln_g, ln_b, Wq, bq, Wg, bg,
           Wf1, bf1, Wf2, bf2):
    B, S, Hd = hidden_states.shape
    n_chunks = (S - CHUNK) // STRIDE + 1
    f32 = jnp.float32

    rows = B * _POOL_ROWS

    PT = 512
    idx16, wexp, vproj = pl.pallas_call(
        functools.partial(_retrieve_body, n_chunks=n_chunks, batch=B,
                          seq_len=S, seq_tile=PT),
        grid=(B, S // PT),
        in_specs=[
            pl.BlockSpec((1, PT, Hd), lambda b, t: (b, t, 0)),
            pl.BlockSpec((Hd, Hd), lambda b, t: (0, 0)),
            pl.BlockSpec((1, Hd), lambda b, t: (0, 0)),
            pl.BlockSpec((Hd, Hd // 2), lambda b, t: (0, 0)),
            pl.BlockSpec((1, Hd // 2), lambda b, t: (0, 0)),
            pl.BlockSpec((Hd // 2, EMB), lambda b, t: (0, 0)),
            pl.BlockSpec((1, EMB), lambda b, t: (0, 0)),
            pl.BlockSpec((1, EMB), lambda b, t: (0, 0)),
            pl.BlockSpec((1, EMB), lambda b, t: (0, 0)),
            pl.BlockSpec((Hd, Hd), lambda b, t: (1, 0)),
            pl.BlockSpec((1, Hd), lambda b, t: (0, 0)),
        ],
        out_specs=(pl.BlockSpec((B, _IDXW), lambda b, t: (0, 0)),
                   pl.BlockSpec((B, _IDXW, 16), lambda b, t: (0, 0, 0)),
                   pl.BlockSpec((rows, _PW), lambda b, t: (0, 0))),
        out_shape=(jax.ShapeDtypeStruct((B, _IDXW), jnp.int32),
                   jax.ShapeDtypeStruct((B, _IDXW, 16), f32),
                   jax.ShapeDtypeStruct((rows, _PW), f32)),
        scratch_shapes=[pltpu.VMEM((B, _POOL_ROWS, Hd), f32)],
    )(hidden_states, Wq, bq.reshape(1, Hd), W1, b1.reshape(1, -1), W2,
      b2.reshape(1, -1), ln_g.reshape(1, -1), ln_b.reshape(1, -1),
      Wf1, Wg[Hd:, 0].reshape(1, Hd))

    sc_kernel = pl.kernel(
        functools.partial(_sc_gather_body, batch=B),
        out_type=(jax.ShapeDtypeStruct((B, Hd), f32),
                  jax.ShapeDtypeStruct((B, 128), f32)),
        mesh=plsc.VectorSubcoreMesh(core_axis_name="c", subcore_axis_name="s"),
        scratch_types=[pltpu.VMEM((_IDXW,), jnp.int32),
                       pltpu.VMEM((_IDXW, 16), f32),
                       pltpu.VMEM((_IDXW, _PW), f32),
                       pltpu.VMEM((_PW,), f32)],
    )
    f1add, gadd = sc_kernel(vproj, idx16, wexp)

    TS = 512
    grid = (B, S // TS)
    out = pl.pallas_call(
        _fusion_body,
        grid=grid,
        in_specs=[
            pl.BlockSpec((1, TS, Hd), lambda b, t: (b, t, 0)),
            pl.BlockSpec((1, 1, Hd), lambda b, t: (b, 0, 0)),
            pl.BlockSpec((1, Hd), lambda b, t: (0, 0)),
            pl.BlockSpec((1, 1, 128), lambda b, t: (b, 0, 0)),
            pl.BlockSpec((1, Hd), lambda b, t: (0, 0)),
            pl.BlockSpec((Hd, Hd), lambda b, t: (0, 0)),
            pl.BlockSpec((Hd, Hd), lambda b, t: (0, 0)),
            pl.BlockSpec((1, Hd), lambda b, t: (0, 0)),
            pl.BlockSpec((1, 128), lambda b, t: (0, 0)),
        ],
        out_specs=pl.BlockSpec((1, TS, Hd), lambda b, t: (b, t, 0)),
        out_shape=jax.ShapeDtypeStruct((B, S, Hd), f32),
        scratch_shapes=[pltpu.VMEM((Hd, Hd), jnp.bfloat16)],
    )(hidden_states, f1add[:, None, :], bf1.reshape(1, Hd),
      gadd[:, None, :], Wg[:Hd, 0].reshape(1, Hd), Wf1,
      Wf2.astype(jnp.bfloat16), bf2.reshape(1, Hd),
      jnp.broadcast_to(bg.reshape(1, 1), (1, 128)))
    return out


# speedup vs baseline: 1.0203x; 1.0005x over previous
import functools

import jax
import jax.numpy as jnp
from jax.experimental import pallas as pl
from jax.experimental.pallas import tpu as pltpu
from jax.experimental.pallas import tpu_sc as plsc

H = 2048
EMB = 128
CHUNK = 64
OVERLAP = 16
TOPK = 3
STRIDE = CHUNK - OVERLAP

_POOL_ROWS = 64
_IDXW = 4
_PW = H + 128

_INV_SQRT2 = 0.7071067811865476


def _gelu_exact(x):
    return 0.5 * x * (1.0 + jax.lax.erf(x * _INV_SQRT2))


def _retrieve_body(hs_ref, wq_ref, bq_ref, w1_ref, b1_ref, w2_ref, b2_ref,
                   lng_ref, lnb_ref, wf1b_ref, wgb_ref,
                   idx_ref, wexp_ref, vproj_ref, x_sc, *,
                   n_chunks, batch, seq_len, seq_tile):
    b, t = pl.program_id(0), pl.program_id(1)
    n_t = pl.num_programs(1)
    hsb = hs_ref[0]
    c = jax.lax.broadcasted_iota(jnp.int32, (_POOL_ROWS, seq_tile), 0)
    s = jax.lax.broadcasted_iota(jnp.int32, (_POOL_ROWS, seq_tile), 1)
    s = s + t * seq_tile
    in_win = (s >= c * STRIDE) & (s < c * STRIDE + CHUNK) & (c < n_chunks)
    pmat = jnp.where(in_win, 1.0 / CHUNK, 0.0)
    pmat = pmat + jnp.where(c == n_chunks, 1.0 / seq_len, 0.0)
    part = jnp.dot(pmat, hsb, preferred_element_type=jnp.float32)

    @pl.when(t == 0)
    def _init():
        x_sc[b] = part

    @pl.when(t != 0)
    def _acc():
        x_sc[b] += part

    @pl.when((b == batch - 1) & (t == n_t - 1))
    def _retrieve():
        _retrieve_tail(wq_ref, bq_ref, w1_ref, b1_ref, w2_ref, b2_ref,
                       lng_ref, lnb_ref, wf1b_ref, wgb_ref,
                       idx_ref, wexp_ref, vproj_ref, x_sc,
                       n_chunks=n_chunks, batch=batch)


def _retrieve_tail(wq_ref, bq_ref, w1_ref, b1_ref, w2_ref, b2_ref,
                   lng_ref, lnb_ref, wf1b_ref, wgb_ref,
                   idx_ref, wexp_ref, vproj_ref, x_sc, *,
                   n_chunks, batch):
    X = x_sc[...]
    rows = batch * _POOL_ROWS
    Xf = X.reshape(rows, H)
    hmean = X[:, n_chunks, :]
    qpool = jnp.dot(hmean, wq_ref[...], preferred_element_type=jnp.float32)
    qpool = qpool + bq_ref[...]
    row_i = jax.lax.broadcasted_iota(jnp.int32, (rows, H), 0)
    qexp = jnp.broadcast_to(qpool[:, None, :], (batch, _POOL_ROWS, H))
    Xe = jnp.where((row_i % _POOL_ROWS) == n_chunks, qexp.reshape(rows, H), Xf)
    h = jnp.dot(Xe, w1_ref[...], preferred_element_type=jnp.float32)
    h = _gelu_exact(h + b1_ref[...])
    e = jnp.dot(h, w2_ref[...], preferred_element_type=jnp.float32) + b2_ref[...]
    mu = jnp.mean(e, axis=-1, keepdims=True)
    var = jnp.mean((e - mu) * (e - mu), axis=-1, keepdims=True)
    e = (e - mu) * jax.lax.rsqrt(var + 1e-5) * lng_ref[...] + lnb_ref[...]
    nrm = jnp.maximum(jnp.sqrt(jnp.sum(e * e, axis=-1, keepdims=True)), 1e-12)
    en = e / nrm
    qe = en.reshape(batch, _POOL_ROWS, EMB)[:, n_chunks, :]
    sim = jax.lax.dot_general(qe, en, (((1,), (1,)), ((), ())),
                              preferred_element_type=jnp.float32)
    col = jax.lax.broadcasted_iota(jnp.int32, (batch, rows), 1)
    colf = col.astype(jnp.float32)
    s = jnp.where((col % _POOL_ROWS) < n_chunks, sim, -1e30)
    scores, onehots = [], []
    for _ in range(TOPK):
        m = jnp.max(s, axis=1, keepdims=True)
        first = jnp.min(jnp.where(s >= m, colf, 1e9), axis=1, keepdims=True)
        oh = (colf == first).astype(jnp.float32)
        scores.append(m)
        onehots.append(oh)
        s = jnp.where(oh > 0.0, -1e30, s)
    es = [jnp.exp(sc - scores[0]) for sc in scores]
    z = es[0] + es[1] + es[2]
    k16 = jax.lax.broadcasted_iota(jnp.int32, (batch, _IDXW), 1)
    idxm = jnp.zeros((batch, _IDXW), jnp.float32)
    wmat = jnp.zeros((batch, _IDXW), jnp.float32)
    for k in range(TOPK):
        first_k = jnp.min(jnp.where(onehots[k] > 0.0, colf, 1e9), axis=1,
                          keepdims=True)
        idxm = jnp.where(k16 == k, first_k, idxm)
        wmat = jnp.where(k16 == k, es[k] / z, wmat)
    idx_ref[...] = idxm.astype(jnp.int32)
    wexp_ref[...] = jnp.broadcast_to(wmat[:, :, None], (batch, _IDXW, 16))
    vproj = jnp.dot(Xf.astype(jnp.bfloat16),
                    wf1b_ref[...].astype(jnp.bfloat16),
                    preferred_element_type=jnp.float32)
    vproj_ref[:, :H] = vproj
    gp = jnp.sum(Xf * wgb_ref[...], axis=1, keepdims=True)
    vproj_ref[:, H:] = jnp.broadcast_to(gp, (rows, 128))


def _sc_gather_body(vproj_hbm, idx_hbm, wexp_hbm, f1_hbm, gadd_hbm,
                    idx_vmem, wexp_vmem, rows_vmem, acc_vmem, sem_a, sem_b,
                    *, batch):
    c = jax.lax.axis_index("c")
    s = jax.lax.axis_index("s")

    @pl.when((c == 0) & (s < batch))
    def _():
        pltpu.async_copy(idx_hbm.at[s], idx_vmem, sem_a).wait()
        gat = pltpu.async_copy(vproj_hbm.at[idx_vmem], rows_vmem, sem_a)
        wcp = pltpu.async_copy(wexp_hbm.at[s], wexp_vmem, sem_b)
        gat.wait()
        wcp.wait()

        @pl.loop(0, H + 128, step=16)
        def _f1(c0):
            sl = pl.ds(c0, 16)
            acc = wexp_vmem[0, :] * rows_vmem[0, sl]
            for k in range(1, TOPK):
                acc += wexp_vmem[k, :] * rows_vmem[k, sl]
            acc_vmem[sl] = acc

        o1 = pltpu.async_copy(acc_vmem.at[pl.ds(0, H)], f1_hbm.at[s], sem_a)
        o2 = pltpu.async_copy(acc_vmem.at[pl.ds(H, 128)], gadd_hbm.at[s],
                              sem_b)
        o1.wait()
        o2.wait()


def _fusion_body(hs_ref, f1_ref, bf1_ref, gadd_ref, wgt_ref, wf1_ref, wf2_ref,
                 bf2_ref, bg_ref, out_ref, wf1bf_ref):
    b, t = pl.program_id(0), pl.program_id(1)

    @pl.when((b == 0) & (t == 0))
    def _cast_weights():
        wf1bf_ref[...] = wf1_ref[...].astype(jnp.bfloat16)

    x = hs_ref[0]
    a = jnp.dot(x.astype(jnp.bfloat16), wf1bf_ref[...],
                preferred_element_type=jnp.float32)
    a = a + f1_ref[0] + bf1_ref[...]
    hgelu = _gelu_exact(a)
    f = jnp.dot(hgelu.astype(jnp.bfloat16), wf2_ref[...],
                preferred_element_type=jnp.float32) + bf2_ref[...]
    gl = (jnp.sum(x * wgt_ref[...], axis=1, keepdims=True)
          + gadd_ref[0][0, 0] + bg_ref[0, 0])
    out_ref[0] = x + jax.nn.sigmoid(gl) * f


def kernel(hidden_states, W1, b1, W2, b2, ln_g, ln_b, Wq, bq, Wg, bg,
           Wf1, bf1, Wf2, bf2):
    B, S, Hd = hidden_states.shape
    n_chunks = (S - CHUNK) // STRIDE + 1
    f32 = jnp.float32

    rows = B * _POOL_ROWS

    PT = 512
    idx16, wexp, vproj = pl.pallas_call(
        functools.partial(_retrieve_body, n_chunks=n_chunks, batch=B,
                          seq_len=S, seq_tile=PT),
        grid=(B, S // PT),
        in_specs=[
            pl.BlockSpec((1, PT, Hd), lambda b, t: (b, t, 0)),
            pl.BlockSpec((Hd, Hd), lambda b, t: (0, 0)),
            pl.BlockSpec((1, Hd), lambda b, t: (0, 0)),
            pl.BlockSpec((Hd, Hd // 2), lambda b, t: (0, 0)),
            pl.BlockSpec((1, Hd // 2), lambda b, t: (0, 0)),
            pl.BlockSpec((Hd // 2, EMB), lambda b, t: (0, 0)),
            pl.BlockSpec((1, EMB), lambda b, t: (0, 0)),
            pl.BlockSpec((1, EMB), lambda b, t: (0, 0)),
            pl.BlockSpec((1, EMB), lambda b, t: (0, 0)),
            pl.BlockSpec((Hd, Hd), lambda b, t: (1, 0)),
            pl.BlockSpec((1, Hd), lambda b, t: (0, 0)),
        ],
        out_specs=(pl.BlockSpec((B, _IDXW), lambda b, t: (0, 0)),
                   pl.BlockSpec((B, _IDXW, 16), lambda b, t: (0, 0, 0)),
                   pl.BlockSpec((rows, _PW), lambda b, t: (0, 0))),
        out_shape=(jax.ShapeDtypeStruct((B, _IDXW), jnp.int32),
                   jax.ShapeDtypeStruct((B, _IDXW, 16), f32),
                   jax.ShapeDtypeStruct((rows, _PW), f32)),
        scratch_shapes=[pltpu.VMEM((B, _POOL_ROWS, Hd), f32)],
    )(hidden_states, Wq, bq.reshape(1, Hd), W1, b1.reshape(1, -1), W2,
      b2.reshape(1, -1), ln_g.reshape(1, -1), ln_b.reshape(1, -1),
      Wf1, Wg[Hd:, 0].reshape(1, Hd))

    sc_kernel = pl.kernel(
        functools.partial(_sc_gather_body, batch=B),
        out_type=(jax.ShapeDtypeStruct((B, Hd), f32),
                  jax.ShapeDtypeStruct((B, 128), f32)),
        mesh=plsc.VectorSubcoreMesh(core_axis_name="c", subcore_axis_name="s"),
        scratch_types=[pltpu.VMEM((_IDXW,), jnp.int32),
                       pltpu.VMEM((_IDXW, 16), f32),
                       pltpu.VMEM((_IDXW, _PW), f32),
                       pltpu.VMEM((_PW,), f32),
                       pltpu.SemaphoreType.DMA,
                       pltpu.SemaphoreType.DMA],
    )
    f1add, gadd = sc_kernel(vproj, idx16, wexp)

    TS = 512
    grid = (B, S // TS)
    out = pl.pallas_call(
        _fusion_body,
        grid=grid,
        in_specs=[
            pl.BlockSpec((1, TS, Hd), lambda b, t: (b, t, 0)),
            pl.BlockSpec((1, 1, Hd), lambda b, t: (b, 0, 0)),
            pl.BlockSpec((1, Hd), lambda b, t: (0, 0)),
            pl.BlockSpec((1, 1, 128), lambda b, t: (b, 0, 0)),
            pl.BlockSpec((1, Hd), lambda b, t: (0, 0)),
            pl.BlockSpec((Hd, Hd), lambda b, t: (0, 0)),
            pl.BlockSpec((Hd, Hd), lambda b, t: (0, 0)),
            pl.BlockSpec((1, Hd), lambda b, t: (0, 0)),
            pl.BlockSpec((1, 128), lambda b, t: (0, 0)),
        ],
        out_specs=pl.BlockSpec((1, TS, Hd), lambda b, t: (b, t, 0)),
        out_shape=jax.ShapeDtypeStruct((B, S, Hd), f32),
        scratch_shapes=[pltpu.VMEM((Hd, Hd), jnp.bfloat16)],
    )(hidden_states, f1add[:, None, :], bf1.reshape(1, Hd),
      gadd[:, None, :], Wg[:Hd, 0].reshape(1, Hd), Wf1,
      Wf2.astype(jnp.bfloat16), bf2.reshape(1, Hd),
      jnp.broadcast_to(bg.reshape(1, 1), (1, 128)))
    return out
